# Initial kernel scaffold; baseline (speedup 1.0000x reference)
import jax
import jax.numpy as jnp
from jax.experimental import pallas as pl


def kernel(t, knots, a, b, c, d):
    raise NotImplementedError("write your pallas kernel here")


# speedup vs baseline: 2.5005x; 2.5005x over previous
import functools

import jax
import jax.numpy as jnp
from jax import lax
from jax.experimental import pallas as pl
from jax.experimental.pallas import tpu as pltpu
from jax.experimental.pallas import tpu_sc as plsc

N_KNOTS = 2048
N_BIN = N_KNOTS - 1
CHANNELS = 512
N_QUERY = 32768

NC = 2
NS = 16
NW = NC * NS
L = 16

QPW = N_QUERY // NW
CQ = L
NCHUNK = QPW // CQ
NJ = CHANNELS // L


def _spline_body(t_hbm, knots_hbm, a_hbm, b_hbm, c_hbm, d_hbm, out_hbm,
                 t_v, idx_v, frac_v, knots_v,
                 ra, rb, rc, rd, out_v, gsem, osem):
    wid = lax.axis_index("s") * NC + lax.axis_index("c")
    base = wid * QPW

    pltpu.sync_copy(t_hbm.at[pl.ds(base, QPW)], t_v)
    pltpu.sync_copy(knots_hbm, knots_v)

    k0 = jnp.full((L,), knots_v[pl.ds(0, L)][0])
    rng = jnp.full((L,), knots_v[pl.ds(N_KNOTS - L, L)][L - 1]) - k0
    nbin = jnp.full((L,), jnp.float32(N_BIN))
    factor = nbin / rng
    step = rng / nbin

    def idx_step(i, _):
        tv = t_v[pl.ds(i * L, L)]
        iv = (factor * tv).astype(jnp.int32)
        idx_v[pl.ds(i * L, L)] = iv
        frac_v[pl.ds(i * L, L)] = tv - (k0 + iv.astype(jnp.float32) * step)
        return 0

    lax.fori_loop(0, QPW // L, idx_step, 0)

    def chunk_step(g, _):
        iv = idx_v[pl.ds(g * CQ, CQ)]
        cpa = pltpu.async_copy(a_hbm.at[iv], ra, gsem)
        cpb = pltpu.async_copy(b_hbm.at[iv], rb, gsem)
        cpc = pltpu.async_copy(c_hbm.at[iv], rc, gsem)
        cpd = pltpu.async_copy(d_hbm.at[iv], rd, gsem)
        fv = frac_v[pl.ds(g * CQ, CQ)]
        fs = [jnp.full((L,), fv[q]) for q in range(CQ)]
        cpa.wait()
        cpb.wait()
        cpc.wait()
        cpd.wait()

        def j_step(j, _):
            s = pl.ds(j * L, L)
            for q in range(CQ):
                f = fs[q]
                out_v[q, s] = ra[q, s] + f * (rb[q, s] + f * (rc[q, s] + f * rd[q, s]))
            return 0

        lax.fori_loop(0, NJ, j_step, 0)
        pltpu.sync_copy(out_v, out_hbm.at[pl.ds(base + g * CQ, CQ)])
        return 0

    lax.fori_loop(0, NCHUNK, chunk_step, 0)


@jax.jit
def kernel(t, knots, a, b, c, d):
    mesh = plsc.VectorSubcoreMesh(core_axis_name="c", subcore_axis_name="s")
    run = pl.kernel(
        _spline_body,
        out_type=jax.ShapeDtypeStruct((N_QUERY, CHANNELS), jnp.float32),
        mesh=mesh,
        scratch_types=[
            pltpu.VMEM((QPW,), jnp.float32),
            pltpu.VMEM((QPW,), jnp.int32),
            pltpu.VMEM((QPW,), jnp.float32),
            pltpu.VMEM((N_KNOTS,), jnp.float32),
            pltpu.VMEM((CQ, CHANNELS), jnp.float32),
            pltpu.VMEM((CQ, CHANNELS), jnp.float32),
            pltpu.VMEM((CQ, CHANNELS), jnp.float32),
            pltpu.VMEM((CQ, CHANNELS), jnp.float32),
            pltpu.VMEM((CQ, CHANNELS), jnp.float32),
            pltpu.SemaphoreType.DMA,
            pltpu.SemaphoreType.DMA,
        ],
    )
    return run(t, knots, a, b, c, d)


# speedup vs baseline: 4.1948x; 1.6776x over previous
import jax
import jax.numpy as jnp
from jax import lax
from jax.experimental import pallas as pl
from jax.experimental.pallas import tpu as pltpu
from jax.experimental.pallas import tpu_sc as plsc

N_KNOTS = 2048
N_BIN = N_KNOTS - 1
CHANNELS = 512
N_QUERY = 32768

NC = 2
NS = 16
NW = NC * NS
L = 16

QPW = N_QUERY // NW
CQ = L
NCHUNK = QPW // CQ
NJ = CHANNELS // L


def _spline_body(t_hbm, knots_hbm, a_hbm, b_hbm, c_hbm, d_hbm, out_hbm,
                 t_v, idx_v, frac_v, knots_v,
                 ra, rb, rc, rd, out_v, gsem0, gsem1, osem0, osem1):
    wid = lax.axis_index("s") * NC + lax.axis_index("c")
    base = wid * QPW
    gsems = (gsem0, gsem1)
    osems = (osem0, osem1)

    pltpu.sync_copy(t_hbm.at[pl.ds(base, QPW)], t_v)
    pltpu.sync_copy(knots_hbm, knots_v)

    k0 = jnp.full((L,), knots_v[pl.ds(0, L)][0])
    rng = jnp.full((L,), knots_v[pl.ds(N_KNOTS - L, L)][L - 1]) - k0
    nbin = jnp.full((L,), jnp.float32(N_BIN))
    factor = nbin / rng
    step = rng / nbin

    def idx_step(i, _):
        tv = t_v[pl.ds(i * L, L)]
        iv = (factor * tv).astype(jnp.int32)
        idx_v[pl.ds(i * L, L)] = iv
        frac_v[pl.ds(i * L, L)] = tv - (k0 + iv.astype(jnp.float32) * step)
        return 0

    lax.fori_loop(0, QPW // L, idx_step, 0)

    def fire_gather(g, b):
        iv = idx_v[pl.ds(g * CQ, CQ)]
        pltpu.async_copy(a_hbm.at[iv], ra.at[b], gsems[b])
        pltpu.async_copy(b_hbm.at[iv], rb.at[b], gsems[b])
        pltpu.async_copy(c_hbm.at[iv], rc.at[b], gsems[b])
        pltpu.async_copy(d_hbm.at[iv], rd.at[b], gsems[b])

    def wait_gather(g, b):
        iv = idx_v[pl.ds(g * CQ, CQ)]
        pltpu.make_async_copy(a_hbm.at[iv], ra.at[b], gsems[b]).wait()
        pltpu.make_async_copy(b_hbm.at[iv], rb.at[b], gsems[b]).wait()
        pltpu.make_async_copy(c_hbm.at[iv], rc.at[b], gsems[b]).wait()
        pltpu.make_async_copy(d_hbm.at[iv], rd.at[b], gsems[b]).wait()

    def out_slice(g):
        return out_hbm.at[pl.ds(base + g * CQ, CQ)]

    def compute(g, b):
        fv = frac_v[pl.ds(g * CQ, CQ)]
        fs = [jnp.full((L,), fv[q]) for q in range(CQ)]

        def j_step(j, _):
            s = pl.ds(j * L, L)
            for q in range(CQ):
                f = fs[q]
                out_v[b, q, s] = (
                    ra[b, q, s]
                    + f * (rb[b, q, s] + f * (rc[b, q, s] + f * rd[b, q, s]))
                )
            return 0

        lax.fori_loop(0, NJ, j_step, 0)

    fire_gather(0, 0)
    fire_gather(1, 1)

    def pair_step(gg, _):
        for b in range(2):
            g = 2 * gg + b
            wait_gather(g, b)

            @pl.when(gg > 0)
            def _():
                pltpu.make_async_copy(out_v.at[b], out_slice(g - 2), osems[b]).wait()

            compute(g, b)

            @pl.when(g + 2 < NCHUNK)
            def _():
                fire_gather(g + 2, b)

            pltpu.async_copy(out_v.at[b], out_slice(g), osems[b])
        return 0

    lax.fori_loop(0, NCHUNK // 2, pair_step, 0)

    pltpu.make_async_copy(out_v.at[0], out_slice(NCHUNK - 2), osems[0]).wait()
    pltpu.make_async_copy(out_v.at[1], out_slice(NCHUNK - 1), osems[1]).wait()


@jax.jit
def kernel(t, knots, a, b, c, d):
    mesh = plsc.VectorSubcoreMesh(core_axis_name="c", subcore_axis_name="s")
    run = pl.kernel(
        _spline_body,
        out_type=jax.ShapeDtypeStruct((N_QUERY, CHANNELS), jnp.float32),
        mesh=mesh,
        scratch_types=[
            pltpu.VMEM((QPW,), jnp.float32),
            pltpu.VMEM((QPW,), jnp.int32),
            pltpu.VMEM((QPW,), jnp.float32),
            pltpu.VMEM((N_KNOTS,), jnp.float32),
            pltpu.VMEM((2, CQ, CHANNELS), jnp.float32),
            pltpu.VMEM((2, CQ, CHANNELS), jnp.float32),
            pltpu.VMEM((2, CQ, CHANNELS), jnp.float32),
            pltpu.VMEM((2, CQ, CHANNELS), jnp.float32),
            pltpu.VMEM((2, CQ, CHANNELS), jnp.float32),
            pltpu.SemaphoreType.DMA,
            pltpu.SemaphoreType.DMA,
            pltpu.SemaphoreType.DMA,
            pltpu.SemaphoreType.DMA,
        ],
    )
    return run(t, knots, a, b, c, d)


# speedup vs baseline: 6.1626x; 1.4691x over previous
import jax
import jax.numpy as jnp
from jax import lax
from jax.experimental import pallas as pl
from jax.experimental.pallas import tpu as pltpu
from jax.experimental.pallas import tpu_sc as plsc

N_KNOTS = 2048
N_BIN = N_KNOTS - 1
CHANNELS = 512
N_QUERY = 32768

NC = 2
NS = 16
NW = NC * NS
L = 16

QPW = N_QUERY // NW
CQ = L
NCHUNK = QPW // CQ
NJ = CHANNELS // L


def _spline_body(t_hbm, knots_hbm, a_hbm, b_hbm, c_hbm, d_hbm, out_hbm,
                 t_v, idx_v, frac_v, knots_v,
                 ra, rb, rc, rd, out_v, gsem0, gsem1, osem0, osem1):
    wid = lax.axis_index("s") * NC + lax.axis_index("c")
    base = wid * QPW
    gsems = (gsem0, gsem1)
    osems = (osem0, osem1)

    pltpu.sync_copy(t_hbm.at[pl.ds(base, QPW)], t_v)
    pltpu.sync_copy(knots_hbm, knots_v)

    k0 = jnp.full((L,), knots_v[pl.ds(0, L)][0])
    rng = jnp.full((L,), knots_v[pl.ds(N_KNOTS - L, L)][L - 1]) - k0
    nbin = jnp.full((L,), jnp.float32(N_BIN))
    factor = nbin / rng
    step = rng / nbin

    def idx_step(i, _):
        tv = t_v[pl.ds(i * L, L)]
        iv = (factor * tv).astype(jnp.int32)
        idx_v[pl.ds(i * L, L)] = iv
        frac_v[pl.ds(i * L, L)] = tv - (k0 + iv.astype(jnp.float32) * step)
        return 0

    lax.fori_loop(0, QPW // L, idx_step, 0)

    def fire_gather(g, b):
        iv = idx_v[pl.ds(g * CQ, CQ)]
        pltpu.async_copy(a_hbm.at[iv], ra.at[b], gsems[b])
        pltpu.async_copy(b_hbm.at[iv], rb.at[b], gsems[b])
        pltpu.async_copy(c_hbm.at[iv], rc.at[b], gsems[b])
        pltpu.async_copy(d_hbm.at[iv], rd.at[b], gsems[b])

    def wait_gather(g, b):
        iv = idx_v[pl.ds(g * CQ, CQ)]
        pltpu.make_async_copy(a_hbm.at[iv], ra.at[b], gsems[b]).wait()
        pltpu.make_async_copy(b_hbm.at[iv], rb.at[b], gsems[b]).wait()
        pltpu.make_async_copy(c_hbm.at[iv], rc.at[b], gsems[b]).wait()
        pltpu.make_async_copy(d_hbm.at[iv], rd.at[b], gsems[b]).wait()

    def out_slice(g):
        return out_hbm.at[pl.ds(base + g * CQ, CQ)]

    def compute(g, b):
        fv = frac_v[pl.ds(g * CQ, CQ)]
        fs = [jnp.full((L,), fv[q]) for q in range(CQ)]

        @plsc.parallel_loop(0, NJ, 1, unroll=2)
        def j_step(j):
            s = pl.ds(j * L, L)
            for q in range(CQ):
                f = fs[q]
                out_v[b, q, s] = (
                    ra[b, q, s]
                    + f * (rb[b, q, s] + f * (rc[b, q, s] + f * rd[b, q, s]))
                )

    fire_gather(0, 0)
    fire_gather(1, 1)

    def pair_step(gg, _):
        for b in range(2):
            g = 2 * gg + b
            wait_gather(g, b)

            @pl.when(gg > 0)
            def _():
                pltpu.make_async_copy(out_v.at[b], out_slice(g - 2), osems[b]).wait()

            compute(g, b)

            @pl.when(g + 2 < NCHUNK)
            def _():
                fire_gather(g + 2, b)

            pltpu.async_copy(out_v.at[b], out_slice(g), osems[b])
        return 0

    lax.fori_loop(0, NCHUNK // 2, pair_step, 0)

    pltpu.make_async_copy(out_v.at[0], out_slice(NCHUNK - 2), osems[0]).wait()
    pltpu.make_async_copy(out_v.at[1], out_slice(NCHUNK - 1), osems[1]).wait()


@jax.jit
def kernel(t, knots, a, b, c, d):
    mesh = plsc.VectorSubcoreMesh(core_axis_name="c", subcore_axis_name="s")
    run = pl.kernel(
        _spline_body,
        out_type=jax.ShapeDtypeStruct((N_QUERY, CHANNELS), jnp.float32),
        mesh=mesh,
        scratch_types=[
            pltpu.VMEM((QPW,), jnp.float32),
            pltpu.VMEM((QPW,), jnp.int32),
            pltpu.VMEM((QPW,), jnp.float32),
            pltpu.VMEM((N_KNOTS,), jnp.float32),
            pltpu.VMEM((2, CQ, CHANNELS), jnp.float32),
            pltpu.VMEM((2, CQ, CHANNELS), jnp.float32),
            pltpu.VMEM((2, CQ, CHANNELS), jnp.float32),
            pltpu.VMEM((2, CQ, CHANNELS), jnp.float32),
            pltpu.VMEM((2, CQ, CHANNELS), jnp.float32),
            pltpu.SemaphoreType.DMA,
            pltpu.SemaphoreType.DMA,
            pltpu.SemaphoreType.DMA,
            pltpu.SemaphoreType.DMA,
        ],
    )
    return run(t, knots, a, b, c, d)
